# baseline (speedup 1.0000x reference)
import functools

import jax
import jax.numpy as jnp
from jax import lax
from jax.experimental import pallas as pl
from jax.experimental.pallas import tpu as pltpu
from jax.experimental.pallas import tpu_sc as plsc

D_MODEL = 64
SCALE = 8.0
NUM_CORES = 2
NUM_SUBCORES = 16
NW = NUM_CORES * NUM_SUBCORES
CHUNK = 128


def kernel(x, emb_weight):
    B, T = x.shape
    N = B * T
    per_w = N // NW
    n_chunks = per_w // CHUNK
    assert per_w * NW == N and n_chunks * CHUNK == per_w

    idx2d = x.reshape(NW * n_chunks, CHUNK).astype(jnp.int32)

    mesh = plsc.VectorSubcoreMesh(
        core_axis_name="c",
        subcore_axis_name="s",
        num_cores=NUM_CORES,
        num_subcores=NUM_SUBCORES,
    )

    @functools.partial(
        pl.kernel,
        out_type=jax.ShapeDtypeStruct((N, D_MODEL), jnp.float32),
        mesh=mesh,
        scratch_types=[
            pltpu.VMEM((n_chunks, CHUNK), jnp.int32),
            pltpu.VMEM((CHUNK, D_MODEL), jnp.float32),
            pltpu.SemaphoreType.DMA,
        ],
        compiler_params=pltpu.CompilerParams(use_tc_tiling_on_sc=False),
    )
    def emb_kernel(idx_hbm, table_hbm, out_hbm, idx_v, buf, sem):
        wid = lax.axis_index("s") * NUM_CORES + lax.axis_index("c")
        base = wid * per_w
        pltpu.sync_copy(idx_hbm.at[pl.ds(wid * n_chunks, n_chunks)], idx_v)

        def chunk_body(c, carry):
            pltpu.async_copy(table_hbm.at[idx_v.at[c]], buf, sem).wait()

            def row_body(r, rc):
                for j in range(D_MODEL // 16):
                    sl = pl.ds(j * 16, 16)
                    buf[r, sl] = buf[r, sl] * SCALE
                return rc

            lax.fori_loop(0, CHUNK, row_body, 0)
            pltpu.sync_copy(buf, out_hbm.at[pl.ds(base + c * CHUNK, CHUNK)])
            return carry

        lax.fori_loop(0, n_chunks, chunk_body, 0)

    out = emb_kernel(idx2d, emb_weight)
    return out.reshape(B, T, D_MODEL)


# speedup vs baseline: 1.2068x; 1.2068x over previous
import functools

import jax
import jax.numpy as jnp
from jax import lax
from jax.experimental import pallas as pl
from jax.experimental.pallas import tpu as pltpu
from jax.experimental.pallas import tpu_sc as plsc

D_MODEL = 64
SCALE = 8.0
NUM_CORES = 2
NUM_SUBCORES = 16
NW = NUM_CORES * NUM_SUBCORES
CHUNK = 128
NBUF = 6


def kernel(x, emb_weight):
    B, T = x.shape
    V = emb_weight.shape[0]
    N = B * T
    per_w = N // NW
    n_chunks = per_w // CHUNK
    assert per_w * NW == N and n_chunks * CHUNK == per_w and n_chunks > 2 * NBUF

    idx2d = x.reshape(NW * n_chunks, CHUNK).astype(jnp.int32)
    table_flat = lax.optimization_barrier(emb_weight.reshape(V * D_MODEL))
    table_lin = table_flat.reshape(V, D_MODEL)

    mesh = plsc.VectorSubcoreMesh(
        core_axis_name="c",
        subcore_axis_name="s",
        num_cores=NUM_CORES,
        num_subcores=NUM_SUBCORES,
    )

    @functools.partial(
        pl.kernel,
        out_type=jax.ShapeDtypeStruct((N, D_MODEL), jnp.float32),
        mesh=mesh,
        scratch_types=[
            pltpu.VMEM((n_chunks, CHUNK), jnp.int32),
            pltpu.VMEM((NBUF, CHUNK, D_MODEL), jnp.float32),
            [pltpu.SemaphoreType.DMA] * NBUF,
            [pltpu.SemaphoreType.DMA] * NBUF,
        ],
        compiler_params=pltpu.CompilerParams(use_tc_tiling_on_sc=False),
    )
    def emb_kernel(idx_hbm, table_hbm, out_hbm, idx_v, bufs, gsems, osems):
        wid = lax.axis_index("s") * NUM_CORES + lax.axis_index("c")
        base = wid * per_w
        pltpu.sync_copy(idx_hbm.at[pl.ds(wid * n_chunks, n_chunks)], idx_v)

        def start_gather(c, b):
            pltpu.async_copy(table_hbm.at[idx_v.at[c]], bufs.at[b], gsems[b])

        def wait_gather(b):
            pltpu.make_async_copy(table_hbm.at[idx_v.at[0]], bufs.at[b],
                                  gsems[b]).wait()

        def start_out(c, b):
            pltpu.async_copy(bufs.at[b], out_hbm.at[pl.ds(base + c * CHUNK, CHUNK)],
                             osems[b])

        def wait_out(b):
            pltpu.make_async_copy(bufs.at[b],
                                  out_hbm.at[pl.ds(base, CHUNK)], osems[b]).wait()

        def scale(b):
            def rows(i, carry):
                for dr in range(8):
                    r = i * 8 + dr
                    for j in range(D_MODEL // 16):
                        sl = pl.ds(j * 16, 16)
                        bufs[b, r, sl] = bufs[b, r, sl] * SCALE
                return carry

            lax.fori_loop(0, CHUNK // 8, rows, 0)

        K = NBUF // 2

        for c in range(K):
            start_gather(c, c)

        for b in range(NBUF):
            c = b
            wait_gather(b)
            scale(b)
            start_out(c, b)
            bg = (b + K) % NBUF
            if b >= K:
                wait_out(bg)
            start_gather(c + K, bg)

        n_waves = n_chunks // NBUF

        def wave(o, carry):
            for b in range(NBUF):
                c = o * NBUF + b
                wait_gather(b)
                scale(b)
                start_out(c, b)
                bg = (b + K) % NBUF

                @pl.when(c + K < n_chunks)
                def _():
                    wait_out(bg)
                    start_gather(c + K, bg)

            return carry

        lax.fori_loop(1, n_waves, wave, 0)

        for b in range(n_chunks % NBUF):
            c = n_waves * NBUF + b
            wait_gather(b)
            scale(b)
            start_out(c, b)

        for b in range(NBUF):
            wait_out(b)

    out = emb_kernel(idx2d, table_lin)
    return out.reshape(B, T, D_MODEL)


# speedup vs baseline: 1.6395x; 1.3585x over previous
import functools

import jax
import jax.numpy as jnp
from jax import lax
from jax.experimental import pallas as pl
from jax.experimental.pallas import tpu as pltpu
from jax.experimental.pallas import tpu_sc as plsc

D_MODEL = 64
SCALE = 8.0
NUM_CORES = 2
NUM_SUBCORES = 16
NW = NUM_CORES * NUM_SUBCORES
CHUNK = 128
NBUF = 6


def _retile_table(emb_weight):
    V = emb_weight.shape[0]
    embT = emb_weight.T
    BLK = 4096
    grid = -(-V // (2 * BLK))
    H = grid * BLK

    def body(a_ref, b_ref, out_ref):
        out_ref[:, 0:D_MODEL] = a_ref[...].T
        out_ref[:, D_MODEL:2 * D_MODEL] = b_ref[...].T

    paired = pl.pallas_call(
        body,
        grid=(grid,),
        in_specs=[
            pl.BlockSpec((D_MODEL, BLK), lambda g: (0, 2 * g)),
            pl.BlockSpec(
                (D_MODEL, BLK),
                lambda g: (0, jnp.minimum(2 * g + 1, (V - 1) // BLK)),
            ),
        ],
        out_specs=pl.BlockSpec((BLK, 2 * D_MODEL), lambda g: (g, 0)),
        out_shape=jax.ShapeDtypeStruct((H, 2 * D_MODEL), jnp.float32),
    )(embT, embT)
    return paired.reshape(2 * H, D_MODEL)


def kernel(x, emb_weight):
    B, T = x.shape
    V = emb_weight.shape[0]
    N = B * T
    per_w = N // NW
    n_chunks = per_w // CHUNK
    assert per_w * NW == N and n_chunks * CHUNK == per_w and n_chunks > 2 * NBUF

    xr = (x & ~8191) + 2 * (x & 4095) + ((x >> 12) & 1)
    idx2d = xr.reshape(NW * n_chunks, CHUNK).astype(jnp.int32)
    table_lin = _retile_table(emb_weight)

    mesh = plsc.VectorSubcoreMesh(
        core_axis_name="c",
        subcore_axis_name="s",
        num_cores=NUM_CORES,
        num_subcores=NUM_SUBCORES,
    )

    @functools.partial(
        pl.kernel,
        out_type=jax.ShapeDtypeStruct((N, D_MODEL), jnp.float32),
        mesh=mesh,
        scratch_types=[
            pltpu.VMEM((n_chunks, CHUNK), jnp.int32),
            pltpu.VMEM((NBUF, CHUNK, D_MODEL), jnp.float32),
            [pltpu.SemaphoreType.DMA] * NBUF,
            [pltpu.SemaphoreType.DMA] * NBUF,
        ],
        compiler_params=pltpu.CompilerParams(use_tc_tiling_on_sc=False),
    )
    def emb_kernel(idx_hbm, table_hbm, out_hbm, idx_v, bufs, gsems, osems):
        wid = lax.axis_index("s") * NUM_CORES + lax.axis_index("c")
        base = wid * per_w
        pltpu.sync_copy(idx_hbm.at[pl.ds(wid * n_chunks, n_chunks)], idx_v)

        def start_gather(c, b):
            pltpu.async_copy(table_hbm.at[idx_v.at[c]], bufs.at[b], gsems[b])

        def wait_gather(b):
            pltpu.make_async_copy(table_hbm.at[idx_v.at[0]], bufs.at[b],
                                  gsems[b]).wait()

        def start_out(c, b):
            pltpu.async_copy(bufs.at[b], out_hbm.at[pl.ds(base + c * CHUNK, CHUNK)],
                             osems[b])

        def wait_out(b):
            pltpu.make_async_copy(bufs.at[b],
                                  out_hbm.at[pl.ds(base, CHUNK)], osems[b]).wait()

        def scale(b):
            def rows(i, carry):
                for dr in range(8):
                    r = i * 8 + dr
                    for j in range(D_MODEL // 16):
                        sl = pl.ds(j * 16, 16)
                        bufs[b, r, sl] = bufs[b, r, sl] * SCALE
                return carry

            lax.fori_loop(0, CHUNK // 8, rows, 0)

        K = NBUF // 2

        for c in range(K):
            start_gather(c, c)

        for b in range(NBUF):
            c = b
            wait_gather(b)
            scale(b)
            start_out(c, b)
            bg = (b + K) % NBUF
            if b >= K:
                wait_out(bg)
            start_gather(c + K, bg)

        n_waves = n_chunks // NBUF

        def wave(o, carry):
            for b in range(NBUF):
                c = o * NBUF + b
                wait_gather(b)
                scale(b)
                start_out(c, b)
                bg = (b + K) % NBUF

                @pl.when(c + K < n_chunks)
                def _():
                    wait_out(bg)
                    start_gather(c + K, bg)

            return carry

        lax.fori_loop(1, n_waves, wave, 0)

        for b in range(n_chunks % NBUF):
            c = n_waves * NBUF + b
            wait_gather(b)
            scale(b)
            start_out(c, b)

        for b in range(NBUF):
            wait_out(b)

    out = emb_kernel(idx2d, table_lin)
    return out.reshape(B, T, D_MODEL)
